# baseline (speedup 1.0000x reference)
import jax
import jax.numpy as jnp
from jax import lax
from jax.experimental import pallas as pl
from jax.experimental.pallas import tpu as pltpu

_CACHE = 512


def _index_body(id_ref, so_ref, dh_ref, pos_ref, nid_ref, rf_ref):
    K = pos_ref.shape[0]
    N = id_ref.shape[1]
    id_row = id_ref[...]
    so_row = so_ref[...]
    dh_row = dh_ref[...]

    kvec = lax.broadcasted_iota(jnp.int32, (K, 1), 0).astype(jnp.float32)
    targets = kvec + (N - K)
    eq = (so_row == targets).astype(jnp.float32)
    cid = jnp.sum(eq * id_row, axis=1, keepdims=True)

    eqr = (cid == (dh_row - 1.0)).astype(jnp.float32)
    reset = jnp.max(eqr)

    iota_n = lax.broadcasted_iota(jnp.int32, (1, N), 1).astype(jnp.float32)
    eq2 = (id_row == cid).astype(jnp.float32)
    pos = jnp.sum(eq2 * iota_n, axis=1, keepdims=True)

    new_id = jnp.where(reset > 0.0, jnp.zeros_like(cid), cid)

    pos_ref[...] = jnp.broadcast_to(pos, pos_ref.shape).astype(jnp.int32)
    nid_ref[...] = jnp.broadcast_to(new_id, nid_ref.shape).astype(jnp.int32)
    rf_ref[0, 0] = (reset > 0.0).astype(jnp.int32)


def _gather_body(pos_ref, rf_ref, h_ref, m_ref, oh_ref, om_ref):
    rst = rf_ref[0] != 0
    oh_ref[...] = jnp.where(rst, jnp.zeros_like(h_ref[...]), h_ref[...])
    om_ref[...] = jnp.where(rst, jnp.zeros_like(m_ref[...]), m_ref[...])


def kernel(id, h, h_padding_mask, sort_order, doc_heads):
    N = id.shape[0]
    T, _, D = h.shape
    H = doc_heads.shape[0]
    K = _CACHE

    id_f = id.astype(jnp.float32).reshape(1, N)
    so_f = sort_order.astype(jnp.float32).reshape(1, N)
    dh_f = doc_heads.astype(jnp.float32).reshape(1, H)

    pos_b, nid_b, rf = pl.pallas_call(
        _index_body,
        in_specs=[
            pl.BlockSpec((1, N), lambda: (0, 0)),
            pl.BlockSpec((1, N), lambda: (0, 0)),
            pl.BlockSpec((1, H), lambda: (0, 0)),
        ],
        out_specs=[
            pl.BlockSpec((K, 128), lambda: (0, 0)),
            pl.BlockSpec((K, 128), lambda: (0, 0)),
            pl.BlockSpec(memory_space=pltpu.SMEM),
        ],
        out_shape=[
            jax.ShapeDtypeStruct((K, 128), jnp.int32),
            jax.ShapeDtypeStruct((K, 128), jnp.int32),
            jax.ShapeDtypeStruct((1, 1), jnp.int32),
        ],
    )(id_f, so_f, dh_f)

    pos = pos_b[:, 0]
    new_id = nid_b[:, 0]
    rflag = rf.reshape(1)

    h2 = h.reshape(T, N * D)
    m3 = h_padding_mask.reshape(N, 1, T)

    oh2, om3 = pl.pallas_call(
        _gather_body,
        grid_spec=pltpu.PrefetchScalarGridSpec(
            num_scalar_prefetch=2,
            grid=(K,),
            in_specs=[
                pl.BlockSpec((T, D), lambda k, pos_r, rf_r: (0, pos_r[k])),
                pl.BlockSpec((1, 1, T), lambda k, pos_r, rf_r: (pos_r[k], 0, 0)),
            ],
            out_specs=[
                pl.BlockSpec((T, D), lambda k, pos_r, rf_r: (0, k)),
                pl.BlockSpec((1, 1, T), lambda k, pos_r, rf_r: (k, 0, 0)),
            ],
        ),
        out_shape=[
            jax.ShapeDtypeStruct((T, K * D), jnp.float32),
            jax.ShapeDtypeStruct((K, 1, T), jnp.float32),
        ],
        compiler_params=pltpu.CompilerParams(
            dimension_semantics=("arbitrary",),
        ),
    )(pos, rflag, h2, m3)

    new_h = oh2.reshape(T, K, D)
    new_mask = om3.reshape(K, T)
    return new_id, new_h, new_mask
